# baseline (speedup 1.0000x reference)
import functools

import jax
import jax.numpy as jnp
from jax import lax
from jax.experimental import pallas as pl
from jax.experimental.pallas import tpu as pltpu
from jax.experimental.pallas import tpu_sc as plsc

N = 10000
E = 320000
D = 128
NH = 2
E2 = E // NH

NC = 2
NS = 16
NW = NC * NS
EPW = E2 // NW
CHUNK = 200
NCHUNK = EPW // CHUNK


def _bdot(a, b):
    return jnp.dot(a.astype(jnp.bfloat16), b.astype(jnp.bfloat16),
                   preferred_element_type=jnp.float32)


def _node_proj_body(x_ref, wa_ref, wb_ref, pa_ref, pb_ref):
    x = x_ref[...]
    pa_ref[...] = _bdot(x, wa_ref[...])
    pb_ref[...] = _bdot(x, wb_ref[...])


def _node_proj(node_x, wa, wb):
    bn = 1000
    grid = (N // bn,)
    return pl.pallas_call(
        _node_proj_body,
        grid=grid,
        in_specs=[
            pl.BlockSpec((bn, D), lambda i: (i, 0)),
            pl.BlockSpec((D, D), lambda i: (0, 0)),
            pl.BlockSpec((D, D), lambda i: (0, 0)),
        ],
        out_specs=[
            pl.BlockSpec((bn, D), lambda i: (i, 0)),
            pl.BlockSpec((bn, D), lambda i: (i, 0)),
        ],
        out_shape=[
            jax.ShapeDtypeStruct((N, D), jnp.float32),
            jax.ShapeDtypeStruct((N, D), jnp.float32),
        ],
    )(node_x, wa, wb)


def _ln_affine(h, g, b):
    mu = jnp.mean(h, axis=-1, keepdims=True)
    hc = h - mu
    var = jnp.mean(hc * hc, axis=-1, keepdims=True)
    return hc * lax.rsqrt(var + 1e-5) * g + b


def _edge_mlp_body(ga_ref, gb_ref, e_ref, wc_ref, w1_ref, w2_ref,
                   b0_ref, b1_ref, b2_ref, g_ref, beta_ref, out_ref):
    e = e_ref[...]
    h0 = ga_ref[...] + gb_ref[...] + _bdot(e, wc_ref[...]) + b0_ref[...]
    h0 = jnp.maximum(h0, 0.0)
    h1 = jnp.maximum(_bdot(h0, w1_ref[...]) + b1_ref[...], 0.0)
    h2 = _bdot(h1, w2_ref[...]) + b2_ref[...]
    out_ref[...] = e + _ln_affine(h2, g_ref[...], beta_ref[...])


BE = 1600


def _edge_mlp(ga, gb, edge_attr, half, wc, w1, w2, b0, b1, b2, g, beta):
    grid = (E2 // BE,)
    off = half * (E2 // BE)
    row = lambda i: (i, 0)
    erow = lambda i: (i + off, 0)
    fix = lambda i: (0, 0)
    return pl.pallas_call(
        _edge_mlp_body,
        grid=grid,
        in_specs=[
            pl.BlockSpec((BE, D), row),
            pl.BlockSpec((BE, D), row),
            pl.BlockSpec((BE, D), erow),
            pl.BlockSpec((D, D), fix),
            pl.BlockSpec((D, D), fix),
            pl.BlockSpec((D, D), fix),
            pl.BlockSpec((1, D), fix),
            pl.BlockSpec((1, D), fix),
            pl.BlockSpec((1, D), fix),
            pl.BlockSpec((1, D), fix),
            pl.BlockSpec((1, D), fix),
        ],
        out_specs=pl.BlockSpec((BE, D), row),
        out_shape=jax.ShapeDtypeStruct((E2, D), jnp.float32),
    )(ga, gb, edge_attr, wc, w1, w2, b0, b1, b2, g, beta)


def _node_update_body(x_ref, p0_ref, p1_ref,
                      wa_ref, wb_ref, w1_ref, w2_ref,
                      b0_ref, b1_ref, b2_ref, g_ref, beta_ref, out_ref):
    x = x_ref[...]
    agg = p0_ref[...] + p1_ref[...]
    h0 = _bdot(x, wa_ref[...]) + _bdot(agg, wb_ref[...]) + b0_ref[...]
    h0 = jnp.maximum(h0, 0.0)
    h1 = jnp.maximum(_bdot(h0, w1_ref[...]) + b1_ref[...], 0.0)
    h2 = _bdot(h1, w2_ref[...]) + b2_ref[...]
    out_ref[...] = x + _ln_affine(h2, g_ref[...], beta_ref[...])


def _node_update(node_x, ps, wa, wb, w1, w2, b0, b1, b2, g, beta):
    bn = 1000
    grid = (N // bn,)
    row = lambda i: (i, 0)
    fix = lambda i: (0, 0)
    return pl.pallas_call(
        _node_update_body,
        grid=grid,
        in_specs=[pl.BlockSpec((bn, D), row)] * 3 + [
            pl.BlockSpec((D, D), fix),
            pl.BlockSpec((D, D), fix),
            pl.BlockSpec((D, D), fix),
            pl.BlockSpec((D, D), fix),
            pl.BlockSpec((1, D), fix),
            pl.BlockSpec((1, D), fix),
            pl.BlockSpec((1, D), fix),
            pl.BlockSpec((1, D), fix),
            pl.BlockSpec((1, D), fix),
        ],
        out_specs=pl.BlockSpec((bn, D), row),
        out_shape=jax.ShapeDtypeStruct((N, D), jnp.float32),
    )(node_x, *ps, wa, wb, w1, w2, b0, b1, b2, g, beta)


@functools.cache
def _sc_gather_kernel(half):
    mesh = plsc.VectorSubcoreMesh(core_axis_name="c", subcore_axis_name="s",
                                  num_cores=NC, num_subcores=NS)

    @functools.partial(
        pl.kernel,
        out_type=[
            jax.ShapeDtypeStruct((E2, D), jnp.float32),
            jax.ShapeDtypeStruct((E2, D), jnp.float32),
        ],
        mesh=mesh,
        scratch_types=[
            pltpu.VMEM((CHUNK,), jnp.int32),
            pltpu.VMEM((CHUNK,), jnp.int32),
            pltpu.VMEM((CHUNK, D), jnp.float32),
            pltpu.VMEM((CHUNK, D), jnp.float32),
            pltpu.SemaphoreType.DMA,
            pltpu.SemaphoreType.DMA,
            pltpu.SemaphoreType.DMA,
            pltpu.SemaphoreType.DMA,
        ],
    )
    def _sc_gather(pa_hbm, pb_hbm, dst_hbm, src_hbm, ga_hbm, gb_hbm,
                   idx0, idx1, rows0, rows1, gs0, gs1, ws0, ws1):
        wid = lax.axis_index("s") * NC + lax.axis_index("c")
        base0 = wid * EPW
        idx_v = (idx0, idx1)
        rows_v = (rows0, rows1)
        gsem = (gs0, gs1)
        wsem = (ws0, ws1)
        njobs = 2 * NCHUNK
        pend_g = {}
        pend_w = {}

        def job_refs(j):
            base = base0 + (j // 2) * CHUNK
            if j % 2 == 0:
                return dst_hbm, pa_hbm, ga_hbm, base
            return src_hbm, pb_hbm, gb_hbm, base

        for j in range(njobs):
            p = j & 1
            if j >= 2:
                pend_w.pop(j - 2).wait()
            idx_hbm, tbl_hbm, out_hbm, base = job_refs(j)
            pltpu.sync_copy(
                idx_hbm.at[pl.ds(half * E2 + base, CHUNK)], idx_v[p])
            pend_g[j] = pltpu.async_copy(tbl_hbm.at[idx_v[p]], rows_v[p],
                                         gsem[p])
            if j >= 1:
                q = (j - 1) & 1
                _, _, out_prev, base_prev = job_refs(j - 1)
                pend_g.pop(j - 1).wait()
                pend_w[j - 1] = pltpu.async_copy(
                    rows_v[q], out_prev.at[pl.ds(base_prev, CHUNK)], wsem[q])
        j = njobs - 1
        q = j & 1
        _, _, out_last, base_last = job_refs(j)
        pend_g.pop(j).wait()
        pend_w[j] = pltpu.async_copy(
            rows_v[q], out_last.at[pl.ds(base_last, CHUNK)], wsem[q])
        pend_w.pop(j - 1).wait()
        pend_w.pop(j).wait()

    return _sc_gather


NPT = -(-((N + NS - 1) // NS) // 8) * 8
NPAD = NPT * NS
EPW_S = E // NW
CHUNK_S = 80
NCHUNK_S = EPW_S // CHUNK_S


@functools.cache
def _sc_scatter_add_kernel():
    mesh = plsc.VectorSubcoreMesh(core_axis_name="c", subcore_axis_name="s",
                                  num_cores=NC, num_subcores=NS)

    @functools.partial(
        pl.kernel,
        out_type=jax.ShapeDtypeStruct((NC, NPAD, D), jnp.float32),
        mesh=mesh,
        scratch_types=[
            pltpu.VMEM((NCHUNK_S, CHUNK_S), jnp.int32),
            pltpu.VMEM((CHUNK_S, D), jnp.float32),
            pltpu.VMEM((CHUNK_S, D), jnp.float32),
            pltpu.VMEM_SHARED((NPAD, D), jnp.float32),
            pltpu.SemaphoreType.DMA,
            pltpu.SemaphoreType.DMA,
        ],
    )
    def _sc_scatter_add(edge0_hbm, edge1_hbm, dst3_hbm, zeros_hbm, out_hbm,
                        idx_all, rows0, rows1, agg_sh, ls0, ls1):
        cid = lax.axis_index("c")
        sid = lax.axis_index("s")
        wid = cid * NS + sid
        base0 = sid * EPW_S
        pltpu.sync_copy(dst3_hbm.at[wid], idx_all)
        pltpu.sync_copy(zeros_hbm.at[pl.ds(sid * NPT, NPT)],
                        agg_sh.at[pl.ds(sid * NPT, NPT)])
        plsc.subcore_barrier()
        rows_v = (rows0, rows1)
        lsem = (ls0, ls1)

        def run_half(edge_hbm):
            pend = {0: pltpu.async_copy(
                edge_hbm.at[pl.ds(base0, CHUNK_S)], rows_v[0], lsem[0])}
            for c in range(NCHUNK_S):
                p = c & 1
                if c + 1 < NCHUNK_S:
                    pend[c + 1] = pltpu.async_copy(
                        edge_hbm.at[pl.ds(base0 + (c + 1) * CHUNK_S,
                                          CHUNK_S)],
                        rows_v[(c + 1) & 1], lsem[(c + 1) & 1])
                pend.pop(c).wait()
                pltpu.sync_copy(rows_v[p], agg_sh.at[idx_all.at[c]],
                                add=True)

        @pl.when(cid == 0)
        def _():
            run_half(edge0_hbm)

        @pl.when(cid == 1)
        def _():
            run_half(edge1_hbm)

        plsc.subcore_barrier()
        pltpu.sync_copy(agg_sh.at[pl.ds(sid * NPT, NPT)],
                        out_hbm.at[cid, pl.ds(sid * NPT, NPT)])

    return _sc_scatter_add


def kernel(node_x, edge_index, edge_attr,
           mW0, mb0, mW1, mb1, mW2, mb2, mg, mB,
           uW0, ub0, uW1, ub1, uW2, ub2, ug, uB):
    src = edge_index[0]
    dst = edge_index[1]

    pa, pb = _node_proj(node_x, mW0[:D], mW0[D:2 * D])
    zeros = jnp.zeros((NPAD, D), jnp.float32)
    dst3 = dst.reshape(NW, NCHUNK_S, CHUNK_S)
    eb = (mb0.reshape(1, D), mb1.reshape(1, D), mb2.reshape(1, D),
          mg.reshape(1, D), mB.reshape(1, D))

    halves = []
    for h in range(NH):
        ga, gb = _sc_gather_kernel(h)(pa, pb, dst, src)
        halves.append(
            _edge_mlp(ga, gb, edge_attr, h, mW0[2 * D:], mW1, mW2, *eb))
    partials = _sc_scatter_add_kernel()(halves[0], halves[1], dst3, zeros)

    ps = [partials[c, :N] for c in range(NC)]
    out = _node_update(
        node_x, ps, uW0[:D], uW0[D:], uW1, uW2,
        ub0.reshape(1, D), ub1.reshape(1, D), ub2.reshape(1, D),
        ug.reshape(1, D), uB.reshape(1, D))
    return out


# speedup vs baseline: 1.1393x; 1.1393x over previous
import functools

import jax
import jax.numpy as jnp
from jax import lax
from jax.experimental import pallas as pl
from jax.experimental.pallas import tpu as pltpu
from jax.experimental.pallas import tpu_sc as plsc

N = 10000
E = 320000
D = 128
NH = 2
E2 = E // NH

NC = 2
NS = 16
NW = NC * NS
EPW = E2 // NW
CHUNK = 200
NCHUNK = EPW // CHUNK


def _bdot(a, b):
    return jnp.dot(a.astype(jnp.bfloat16), b.astype(jnp.bfloat16),
                   preferred_element_type=jnp.float32)


def _node_proj_body(x_ref, wa_ref, wb_ref, pa_ref, pb_ref):
    x = x_ref[...]
    pa_ref[...] = _bdot(x, wa_ref[...])
    pb_ref[...] = _bdot(x, wb_ref[...])


def _node_proj(node_x, wa, wb):
    bn = 1000
    grid = (N // bn,)
    return pl.pallas_call(
        _node_proj_body,
        grid=grid,
        in_specs=[
            pl.BlockSpec((bn, D), lambda i: (i, 0)),
            pl.BlockSpec((D, D), lambda i: (0, 0)),
            pl.BlockSpec((D, D), lambda i: (0, 0)),
        ],
        out_specs=[
            pl.BlockSpec((bn, D), lambda i: (i, 0)),
            pl.BlockSpec((bn, D), lambda i: (i, 0)),
        ],
        out_shape=[
            jax.ShapeDtypeStruct((N, D), jnp.float32),
            jax.ShapeDtypeStruct((N, D), jnp.float32),
        ],
    )(node_x, wa, wb)


def _ln_affine(h, g, b):
    mu = jnp.mean(h, axis=-1, keepdims=True)
    hc = h - mu
    var = jnp.mean(hc * hc, axis=-1, keepdims=True)
    return hc * lax.rsqrt(var + 1e-5) * g + b


def _edge_mlp_body(gsum_ref, e_ref, wc_ref, w1_ref, w2_ref,
                   b0_ref, b1_ref, b2_ref, g_ref, beta_ref, out_ref):
    e = e_ref[...]
    h0 = gsum_ref[...] + _bdot(e, wc_ref[...]) + b0_ref[...]
    h0 = jnp.maximum(h0, 0.0)
    h1 = jnp.maximum(_bdot(h0, w1_ref[...]) + b1_ref[...], 0.0)
    h2 = _bdot(h1, w2_ref[...]) + b2_ref[...]
    out_ref[...] = e + _ln_affine(h2, g_ref[...], beta_ref[...])


BE = 1600


def _edge_mlp(gsum, edge_attr, half, wc, w1, w2, b0, b1, b2, g, beta):
    grid = (E2 // BE,)
    off = half * (E2 // BE)
    row = lambda i: (i, 0)
    erow = lambda i: (i + off, 0)
    fix = lambda i: (0, 0)
    return pl.pallas_call(
        _edge_mlp_body,
        grid=grid,
        in_specs=[
            pl.BlockSpec((BE, D), row),
            pl.BlockSpec((BE, D), erow),
            pl.BlockSpec((D, D), fix),
            pl.BlockSpec((D, D), fix),
            pl.BlockSpec((D, D), fix),
            pl.BlockSpec((1, D), fix),
            pl.BlockSpec((1, D), fix),
            pl.BlockSpec((1, D), fix),
            pl.BlockSpec((1, D), fix),
            pl.BlockSpec((1, D), fix),
        ],
        out_specs=pl.BlockSpec((BE, D), row),
        out_shape=jax.ShapeDtypeStruct((E2, D), jnp.float32),
    )(gsum, edge_attr, wc, w1, w2, b0, b1, b2, g, beta)


def _node_update_body(x_ref, p0_ref, p1_ref,
                      wa_ref, wb_ref, w1_ref, w2_ref,
                      b0_ref, b1_ref, b2_ref, g_ref, beta_ref, out_ref):
    x = x_ref[...]
    agg = p0_ref[...] + p1_ref[...]
    h0 = _bdot(x, wa_ref[...]) + _bdot(agg, wb_ref[...]) + b0_ref[...]
    h0 = jnp.maximum(h0, 0.0)
    h1 = jnp.maximum(_bdot(h0, w1_ref[...]) + b1_ref[...], 0.0)
    h2 = _bdot(h1, w2_ref[...]) + b2_ref[...]
    out_ref[...] = x + _ln_affine(h2, g_ref[...], beta_ref[...])


def _node_update(node_x, ps, wa, wb, w1, w2, b0, b1, b2, g, beta):
    bn = 1000
    grid = (N // bn,)
    row = lambda i: (i, 0)
    fix = lambda i: (0, 0)
    return pl.pallas_call(
        _node_update_body,
        grid=grid,
        in_specs=[pl.BlockSpec((bn, D), row)] * 3 + [
            pl.BlockSpec((D, D), fix),
            pl.BlockSpec((D, D), fix),
            pl.BlockSpec((D, D), fix),
            pl.BlockSpec((D, D), fix),
            pl.BlockSpec((1, D), fix),
            pl.BlockSpec((1, D), fix),
            pl.BlockSpec((1, D), fix),
            pl.BlockSpec((1, D), fix),
            pl.BlockSpec((1, D), fix),
        ],
        out_specs=pl.BlockSpec((bn, D), row),
        out_shape=jax.ShapeDtypeStruct((N, D), jnp.float32),
    )(node_x, *ps, wa, wb, w1, w2, b0, b1, b2, g, beta)


@functools.cache
def _sc_gather_kernel(half):
    mesh = plsc.VectorSubcoreMesh(core_axis_name="c", subcore_axis_name="s",
                                  num_cores=NC, num_subcores=NS)

    @functools.partial(
        pl.kernel,
        out_type=jax.ShapeDtypeStruct((E2, D), jnp.float32),
        mesh=mesh,
        scratch_types=[
            pltpu.VMEM((CHUNK,), jnp.int32),
            pltpu.VMEM((CHUNK,), jnp.int32),
            pltpu.VMEM((CHUNK,), jnp.int32),
            pltpu.VMEM((CHUNK,), jnp.int32),
            pltpu.VMEM((CHUNK, D), jnp.float32),
            pltpu.VMEM((CHUNK, D), jnp.float32),
            pltpu.SemaphoreType.DMA,
            pltpu.SemaphoreType.DMA,
            pltpu.SemaphoreType.DMA,
            pltpu.SemaphoreType.DMA,
            pltpu.SemaphoreType.DMA,
            pltpu.SemaphoreType.DMA,
        ],
    )
    def _sc_gather(pa_hbm, pb_hbm, dst_hbm, src_hbm, gsum_hbm,
                   ia0, ia1, ib0, ib1, rows0, rows1,
                   as0, as1, bs0, bs1, ws0, ws1):
        wid = lax.axis_index("s") * NC + lax.axis_index("c")
        base0 = half * E2 + wid * EPW
        obase0 = wid * EPW
        idx_a = (ia0, ia1)
        idx_b = (ib0, ib1)
        rows_v = (rows0, rows1)
        asem = (as0, as1)
        bsem = (bs0, bs1)
        wsem = (ws0, ws1)
        pend_a = {}
        pend_b = {}
        pend_w = {}
        for j in range(NCHUNK):
            p = j & 1
            if j >= 2:
                pend_w.pop(j - 2).wait()
            base = base0 + j * CHUNK
            pltpu.sync_copy(dst_hbm.at[pl.ds(base, CHUNK)], idx_a[p])
            pend_a[j] = pltpu.async_copy(pa_hbm.at[idx_a[p]], rows_v[p],
                                         asem[p])
            pltpu.sync_copy(src_hbm.at[pl.ds(base, CHUNK)], idx_b[p])
            pend_a.pop(j).wait()
            pend_b[j] = pltpu.async_copy(pb_hbm.at[idx_b[p]], rows_v[p],
                                         bsem[p], add=True)
            if j >= 1:
                q = (j - 1) & 1
                pend_b.pop(j - 1).wait()
                pend_w[j - 1] = pltpu.async_copy(
                    rows_v[q],
                    gsum_hbm.at[pl.ds(obase0 + (j - 1) * CHUNK, CHUNK)],
                    wsem[q])
        j = NCHUNK - 1
        q = j & 1
        pend_b.pop(j).wait()
        pend_w[j] = pltpu.async_copy(
            rows_v[q], gsum_hbm.at[pl.ds(obase0 + j * CHUNK, CHUNK)],
            wsem[q])
        pend_w.pop(j - 1).wait()
        pend_w.pop(j).wait()

    return _sc_gather


NPT = -(-((N + NS - 1) // NS) // 8) * 8
NPAD = NPT * NS
EPW_S = E // NW
CHUNK_S = 80
NCHUNK_S = EPW_S // CHUNK_S


@functools.cache
def _sc_scatter_add_kernel():
    mesh = plsc.VectorSubcoreMesh(core_axis_name="c", subcore_axis_name="s",
                                  num_cores=NC, num_subcores=NS)

    @functools.partial(
        pl.kernel,
        out_type=jax.ShapeDtypeStruct((NC, NPAD, D), jnp.float32),
        mesh=mesh,
        scratch_types=[
            pltpu.VMEM((NCHUNK_S, CHUNK_S), jnp.int32),
            pltpu.VMEM((CHUNK_S, D), jnp.float32),
            pltpu.VMEM((CHUNK_S, D), jnp.float32),
            pltpu.VMEM_SHARED((NPAD, D), jnp.float32),
            pltpu.SemaphoreType.DMA,
            pltpu.SemaphoreType.DMA,
        ],
    )
    def _sc_scatter_add(edge0_hbm, edge1_hbm, dst3_hbm, zeros_hbm, out_hbm,
                        idx_all, rows0, rows1, agg_sh, ls0, ls1):
        cid = lax.axis_index("c")
        sid = lax.axis_index("s")
        wid = cid * NS + sid
        base0 = sid * EPW_S
        pltpu.sync_copy(dst3_hbm.at[wid], idx_all)
        pltpu.sync_copy(zeros_hbm.at[pl.ds(sid * NPT, NPT)],
                        agg_sh.at[pl.ds(sid * NPT, NPT)])
        plsc.subcore_barrier()
        rows_v = (rows0, rows1)
        lsem = (ls0, ls1)

        def run_half(edge_hbm):
            pend = {0: pltpu.async_copy(
                edge_hbm.at[pl.ds(base0, CHUNK_S)], rows_v[0], lsem[0])}
            for c in range(NCHUNK_S):
                p = c & 1
                if c + 1 < NCHUNK_S:
                    pend[c + 1] = pltpu.async_copy(
                        edge_hbm.at[pl.ds(base0 + (c + 1) * CHUNK_S,
                                          CHUNK_S)],
                        rows_v[(c + 1) & 1], lsem[(c + 1) & 1])
                pend.pop(c).wait()
                pltpu.sync_copy(rows_v[p], agg_sh.at[idx_all.at[c]],
                                add=True)

        @pl.when(cid == 0)
        def _():
            run_half(edge0_hbm)

        @pl.when(cid == 1)
        def _():
            run_half(edge1_hbm)

        plsc.subcore_barrier()
        pltpu.sync_copy(agg_sh.at[pl.ds(sid * NPT, NPT)],
                        out_hbm.at[cid, pl.ds(sid * NPT, NPT)])

    return _sc_scatter_add


def kernel(node_x, edge_index, edge_attr,
           mW0, mb0, mW1, mb1, mW2, mb2, mg, mB,
           uW0, ub0, uW1, ub1, uW2, ub2, ug, uB):
    src = edge_index[0]
    dst = edge_index[1]

    pa, pb = _node_proj(node_x, mW0[:D], mW0[D:2 * D])
    zeros = jnp.zeros((NPAD, D), jnp.float32)
    dst3 = dst.reshape(NW, NCHUNK_S, CHUNK_S)
    eb = (mb0.reshape(1, D), mb1.reshape(1, D), mb2.reshape(1, D),
          mg.reshape(1, D), mB.reshape(1, D))

    halves = []
    for h in range(NH):
        gsum = _sc_gather_kernel(h)(pa, pb, dst, src)
        halves.append(
            _edge_mlp(gsum, edge_attr, h, mW0[2 * D:], mW1, mW2, *eb))
    partials = _sc_scatter_add_kernel()(halves[0], halves[1], dst3, zeros)

    ps = [partials[c, :N] for c in range(NC)]
    out = _node_update(
        node_x, ps, uW0[:D], uW0[D:], uW1, uW2,
        ub0.reshape(1, D), ub1.reshape(1, D), ub2.reshape(1, D),
        ug.reshape(1, D), uB.reshape(1, D))
    return out


# speedup vs baseline: 1.1883x; 1.0431x over previous
import functools

import jax
import jax.numpy as jnp
from jax import lax
from jax.experimental import pallas as pl
from jax.experimental.pallas import tpu as pltpu
from jax.experimental.pallas import tpu_sc as plsc

N = 10000
E = 320000
D = 128
NH = 2
E2 = E // NH

NC = 2
NS = 16
NW = NC * NS
EPW = E2 // NW
CHUNK = 200
NCHUNK = EPW // CHUNK


def _bdot(a, b):
    return jnp.dot(a.astype(jnp.bfloat16), b.astype(jnp.bfloat16),
                   preferred_element_type=jnp.float32)


def _node_proj_body(x_ref, wa_ref, wb_ref, pa_ref, pb_ref):
    x = x_ref[...]
    pa_ref[...] = _bdot(x, wa_ref[...])
    pb_ref[...] = _bdot(x, wb_ref[...])


def _node_proj(node_x, wa, wb):
    bn = 1000
    grid = (N // bn,)
    return pl.pallas_call(
        _node_proj_body,
        grid=grid,
        in_specs=[
            pl.BlockSpec((bn, D), lambda i: (i, 0)),
            pl.BlockSpec((D, D), lambda i: (0, 0)),
            pl.BlockSpec((D, D), lambda i: (0, 0)),
        ],
        out_specs=[
            pl.BlockSpec((bn, D), lambda i: (i, 0)),
            pl.BlockSpec((bn, D), lambda i: (i, 0)),
        ],
        out_shape=[
            jax.ShapeDtypeStruct((N, D), jnp.float32),
            jax.ShapeDtypeStruct((N, D), jnp.float32),
        ],
    )(node_x, wa, wb)


def _ln_affine(h, g, b):
    mu = jnp.mean(h, axis=-1, keepdims=True)
    hc = h - mu
    var = jnp.mean(hc * hc, axis=-1, keepdims=True)
    return hc * lax.rsqrt(var + 1e-5) * g + b


def _edge_mlp_body(gsum_ref, e_ref, wc_ref, w1_ref, w2_ref,
                   b0_ref, b1_ref, b2_ref, g_ref, beta_ref, out_ref):
    e = e_ref[...]
    h0 = gsum_ref[...] + _bdot(e, wc_ref[...]) + b0_ref[...]
    h0 = jnp.maximum(h0, 0.0)
    h1 = jnp.maximum(_bdot(h0, w1_ref[...]) + b1_ref[...], 0.0)
    h2 = _bdot(h1, w2_ref[...]) + b2_ref[...]
    out_ref[...] = e + _ln_affine(h2, g_ref[...], beta_ref[...])


BE = 1600


def _edge_mlp(gsum, edge_attr, half, wc, w1, w2, b0, b1, b2, g, beta):
    grid = (E2 // BE,)
    off = half * (E2 // BE)
    row = lambda i: (i, 0)
    erow = lambda i: (i + off, 0)
    fix = lambda i: (0, 0)
    return pl.pallas_call(
        _edge_mlp_body,
        grid=grid,
        in_specs=[
            pl.BlockSpec((BE, D), row),
            pl.BlockSpec((BE, D), erow),
            pl.BlockSpec((D, D), fix),
            pl.BlockSpec((D, D), fix),
            pl.BlockSpec((D, D), fix),
            pl.BlockSpec((1, D), fix),
            pl.BlockSpec((1, D), fix),
            pl.BlockSpec((1, D), fix),
            pl.BlockSpec((1, D), fix),
            pl.BlockSpec((1, D), fix),
        ],
        out_specs=pl.BlockSpec((BE, D), row),
        out_shape=jax.ShapeDtypeStruct((E2, D), jnp.float32),
    )(gsum, edge_attr, wc, w1, w2, b0, b1, b2, g, beta)


def _node_update_body(x_ref, p0_ref, p1_ref,
                      wa_ref, wb_ref, w1_ref, w2_ref,
                      b0_ref, b1_ref, b2_ref, g_ref, beta_ref, out_ref):
    x = x_ref[...]
    agg = p0_ref[...] + p1_ref[...]
    h0 = _bdot(x, wa_ref[...]) + _bdot(agg, wb_ref[...]) + b0_ref[...]
    h0 = jnp.maximum(h0, 0.0)
    h1 = jnp.maximum(_bdot(h0, w1_ref[...]) + b1_ref[...], 0.0)
    h2 = _bdot(h1, w2_ref[...]) + b2_ref[...]
    out_ref[...] = x + _ln_affine(h2, g_ref[...], beta_ref[...])


def _node_update(node_x, ps, wa, wb, w1, w2, b0, b1, b2, g, beta):
    bn = 1000
    grid = (N // bn,)
    row = lambda i: (i, 0)
    fix = lambda i: (0, 0)
    return pl.pallas_call(
        _node_update_body,
        grid=grid,
        in_specs=[pl.BlockSpec((bn, D), row)] * 3 + [
            pl.BlockSpec((D, D), fix),
            pl.BlockSpec((D, D), fix),
            pl.BlockSpec((D, D), fix),
            pl.BlockSpec((D, D), fix),
            pl.BlockSpec((1, D), fix),
            pl.BlockSpec((1, D), fix),
            pl.BlockSpec((1, D), fix),
            pl.BlockSpec((1, D), fix),
            pl.BlockSpec((1, D), fix),
        ],
        out_specs=pl.BlockSpec((bn, D), row),
        out_shape=jax.ShapeDtypeStruct((N, D), jnp.float32),
    )(node_x, *ps, wa, wb, w1, w2, b0, b1, b2, g, beta)


@functools.cache
def _sc_gather_kernel(half):
    mesh = plsc.VectorSubcoreMesh(core_axis_name="c", subcore_axis_name="s",
                                  num_cores=NC, num_subcores=NS)

    @functools.partial(
        pl.kernel,
        out_type=jax.ShapeDtypeStruct((E2, D), jnp.float32),
        mesh=mesh,
        scratch_types=[
            pltpu.VMEM((EPW,), jnp.int32),
            pltpu.VMEM((EPW,), jnp.int32),
            pltpu.VMEM((CHUNK, D), jnp.float32),
            pltpu.VMEM((CHUNK, D), jnp.float32),
            pltpu.VMEM((CHUNK, D), jnp.float32),
            pltpu.SemaphoreType.DMA,
            pltpu.SemaphoreType.DMA,
            pltpu.SemaphoreType.DMA,
            pltpu.SemaphoreType.DMA,
            pltpu.SemaphoreType.DMA,
            pltpu.SemaphoreType.DMA,
            pltpu.SemaphoreType.DMA,
            pltpu.SemaphoreType.DMA,
            pltpu.SemaphoreType.DMA,
        ],
    )
    def _sc_gather(pa_hbm, pb_hbm, dst_hbm, src_hbm, gsum_hbm,
                   idx_a, idx_b, rows0, rows1, rows2,
                   as0, as1, as2, bs0, bs1, bs2, ws0, ws1, ws2):
        wid = lax.axis_index("s") * NC + lax.axis_index("c")
        base0 = half * E2 + wid * EPW
        obase0 = wid * EPW
        rows_v = (rows0, rows1, rows2)
        asem = (as0, as1, as2)
        bsem = (bs0, bs1, bs2)
        wsem = (ws0, ws1, ws2)
        pltpu.sync_copy(dst_hbm.at[pl.ds(base0, EPW)], idx_a)
        pltpu.sync_copy(src_hbm.at[pl.ds(base0, EPW)], idx_b)
        pend_a = {}
        pend_b = {}
        pend_w = {}

        def start_a(j):
            pend_a[j] = pltpu.async_copy(
                pa_hbm.at[idx_a.at[pl.ds(j * CHUNK, CHUNK)]],
                rows_v[j % 3], asem[j % 3])

        def start_b(j):
            pend_a.pop(j).wait()
            pend_b[j] = pltpu.async_copy(
                pb_hbm.at[idx_b.at[pl.ds(j * CHUNK, CHUNK)]],
                rows_v[j % 3], bsem[j % 3], add=True)

        def start_w(j):
            pend_b.pop(j).wait()
            pend_w[j] = pltpu.async_copy(
                rows_v[j % 3],
                gsum_hbm.at[pl.ds(obase0 + j * CHUNK, CHUNK)], wsem[j % 3])

        for j in range(NCHUNK):
            if j >= 3:
                pend_w.pop(j - 3).wait()
            start_a(j)
            if j >= 1:
                start_b(j - 1)
            if j >= 2:
                start_w(j - 2)
        start_b(NCHUNK - 1)
        start_w(NCHUNK - 2)
        start_w(NCHUNK - 1)
        for j in (NCHUNK - 3, NCHUNK - 2, NCHUNK - 1):
            if j >= 0 and j in pend_w:
                pend_w.pop(j).wait()

    return _sc_gather


NPT = -(-((N + NS - 1) // NS) // 8) * 8
NPAD = NPT * NS
EPW_S = E // NW
CHUNK_S = 80
NCHUNK_S = EPW_S // CHUNK_S


@functools.cache
def _sc_scatter_add_kernel():
    mesh = plsc.VectorSubcoreMesh(core_axis_name="c", subcore_axis_name="s",
                                  num_cores=NC, num_subcores=NS)

    @functools.partial(
        pl.kernel,
        out_type=jax.ShapeDtypeStruct((NC, NPAD, D), jnp.float32),
        mesh=mesh,
        scratch_types=[
            pltpu.VMEM((NCHUNK_S, CHUNK_S), jnp.int32),
            pltpu.VMEM((CHUNK_S, D), jnp.float32),
            pltpu.VMEM((CHUNK_S, D), jnp.float32),
            pltpu.VMEM((CHUNK_S, D), jnp.float32),
            pltpu.VMEM_SHARED((NPAD, D), jnp.float32),
            pltpu.SemaphoreType.DMA,
            pltpu.SemaphoreType.DMA,
            pltpu.SemaphoreType.DMA,
            pltpu.SemaphoreType.DMA,
            pltpu.SemaphoreType.DMA,
            pltpu.SemaphoreType.DMA,
        ],
    )
    def _sc_scatter_add(edge0_hbm, edge1_hbm, dst3_hbm, zeros_hbm, out_hbm,
                        idx_all, rows0, rows1, rows2, agg_sh,
                        ls0, ls1, ls2, as0, as1, as2):
        cid = lax.axis_index("c")
        sid = lax.axis_index("s")
        wid = cid * NS + sid
        base0 = sid * EPW_S
        pltpu.sync_copy(dst3_hbm.at[wid], idx_all)
        pltpu.sync_copy(zeros_hbm.at[pl.ds(sid * NPT, NPT)],
                        agg_sh.at[pl.ds(sid * NPT, NPT)])
        plsc.subcore_barrier()
        rows_v = (rows0, rows1, rows2)
        lsem = (ls0, ls1, ls2)
        addsem = (as0, as1, as2)

        def run_half(edge_hbm):
            pend_l = {}
            pend_add = {}

            def start_l(c):
                pend_l[c] = pltpu.async_copy(
                    edge_hbm.at[pl.ds(base0 + c * CHUNK_S, CHUNK_S)],
                    rows_v[c % 3], lsem[c % 3])

            start_l(0)
            for c in range(NCHUNK_S):
                if c + 1 < NCHUNK_S:
                    if c >= 2:
                        pend_add.pop(c - 2).wait()
                    start_l(c + 1)
                pend_l.pop(c).wait()
                pend_add[c] = pltpu.async_copy(
                    rows_v[c % 3], agg_sh.at[idx_all.at[c]], addsem[c % 3],
                    add=True)
            for c in sorted(pend_add):
                pend_add.pop(c).wait()

        @pl.when(cid == 0)
        def _():
            run_half(edge0_hbm)

        @pl.when(cid == 1)
        def _():
            run_half(edge1_hbm)

        plsc.subcore_barrier()
        pltpu.sync_copy(agg_sh.at[pl.ds(sid * NPT, NPT)],
                        out_hbm.at[cid, pl.ds(sid * NPT, NPT)])

    return _sc_scatter_add


def kernel(node_x, edge_index, edge_attr,
           mW0, mb0, mW1, mb1, mW2, mb2, mg, mB,
           uW0, ub0, uW1, ub1, uW2, ub2, ug, uB):
    src = edge_index[0]
    dst = edge_index[1]

    pa, pb = _node_proj(node_x, mW0[:D], mW0[D:2 * D])
    zeros = jnp.zeros((NPAD, D), jnp.float32)
    dst3s = dst.reshape(NW, NCHUNK_S, CHUNK_S)
    eb = (mb0.reshape(1, D), mb1.reshape(1, D), mb2.reshape(1, D),
          mg.reshape(1, D), mB.reshape(1, D))

    halves = []
    for h in range(NH):
        gsum = _sc_gather_kernel(h)(pa, pb, dst, src)
        halves.append(
            _edge_mlp(gsum, edge_attr, h, mW0[2 * D:], mW1, mW2, *eb))
    partials = _sc_scatter_add_kernel()(halves[0], halves[1], dst3s, zeros)

    ps = [partials[c, :N] for c in range(NC)]
    out = _node_update(
        node_x, ps, uW0[:D], uW0[D:], uW1, uW2,
        ub0.reshape(1, D), ub1.reshape(1, D), ub2.reshape(1, D),
        ug.reshape(1, D), uB.reshape(1, D))
    return out


# speedup vs baseline: 1.3229x; 1.1133x over previous
import functools

import jax
import jax.numpy as jnp
from jax import lax
from jax.experimental import pallas as pl
from jax.experimental.pallas import tpu as pltpu
from jax.experimental.pallas import tpu_sc as plsc

N = 10000
E = 320000
D = 128
NH = 2
E2 = E // NH

NC = 2
NS = 16
NW = NC * NS
EPW = E2 // NW
CHUNK = 200
NCHUNK = EPW // CHUNK


def _bdot(a, b):
    return jnp.dot(a.astype(jnp.bfloat16), b.astype(jnp.bfloat16),
                   preferred_element_type=jnp.float32)


def _node_proj_body(x_ref, wa_ref, wb_ref, pa_ref, pb_ref):
    x = x_ref[...]
    pa_ref[...] = _bdot(x, wa_ref[...])
    pb_ref[...] = _bdot(x, wb_ref[...])


def _node_proj(node_x, wa, wb):
    bn = 1000
    grid = (N // bn,)
    return pl.pallas_call(
        _node_proj_body,
        grid=grid,
        in_specs=[
            pl.BlockSpec((bn, D), lambda i: (i, 0)),
            pl.BlockSpec((D, D), lambda i: (0, 0)),
            pl.BlockSpec((D, D), lambda i: (0, 0)),
        ],
        out_specs=[
            pl.BlockSpec((bn, D), lambda i: (i, 0)),
            pl.BlockSpec((bn, D), lambda i: (i, 0)),
        ],
        out_shape=[
            jax.ShapeDtypeStruct((N, D), jnp.float32),
            jax.ShapeDtypeStruct((N, D), jnp.float32),
        ],
    )(node_x, wa, wb)


def _ln_affine(h, g, b):
    mu = jnp.mean(h, axis=-1, keepdims=True)
    hc = h - mu
    var = jnp.mean(hc * hc, axis=-1, keepdims=True)
    return hc * lax.rsqrt(var + 1e-5) * g + b


def _edge_mlp_body(gsum_ref, e_ref, wc_ref, w1_ref, w2_ref,
                   b0_ref, b1_ref, b2_ref, g_ref, beta_ref, out_ref):
    e = e_ref[...]
    h0 = gsum_ref[...] + _bdot(e, wc_ref[...]) + b0_ref[...]
    h0 = jnp.maximum(h0, 0.0)
    h1 = jnp.maximum(_bdot(h0, w1_ref[...]) + b1_ref[...], 0.0)
    h2 = _bdot(h1, w2_ref[...]) + b2_ref[...]
    out_ref[...] = e + _ln_affine(h2, g_ref[...], beta_ref[...])


BE = 3200


def _edge_mlp(gsum, edge_attr, half, wc, w1, w2, b0, b1, b2, g, beta):
    grid = (E2 // BE,)
    off = half * (E2 // BE)
    row = lambda i: (i, 0)
    erow = lambda i: (i + off, 0)
    fix = lambda i: (0, 0)
    return pl.pallas_call(
        _edge_mlp_body,
        grid=grid,
        in_specs=[
            pl.BlockSpec((BE, D), row),
            pl.BlockSpec((BE, D), erow),
            pl.BlockSpec((D, D), fix),
            pl.BlockSpec((D, D), fix),
            pl.BlockSpec((D, D), fix),
            pl.BlockSpec((1, D), fix),
            pl.BlockSpec((1, D), fix),
            pl.BlockSpec((1, D), fix),
            pl.BlockSpec((1, D), fix),
            pl.BlockSpec((1, D), fix),
        ],
        out_specs=pl.BlockSpec((BE, D), row),
        out_shape=jax.ShapeDtypeStruct((E2, D), jnp.float32),
    )(gsum, edge_attr, wc, w1, w2, b0, b1, b2, g, beta)


def _node_update_body(x_ref, p0_ref, p1_ref,
                      wa_ref, wb_ref, w1_ref, w2_ref,
                      b0_ref, b1_ref, b2_ref, g_ref, beta_ref, out_ref):
    x = x_ref[...]
    agg = p0_ref[...] + p1_ref[...]
    h0 = _bdot(x, wa_ref[...]) + _bdot(agg, wb_ref[...]) + b0_ref[...]
    h0 = jnp.maximum(h0, 0.0)
    h1 = jnp.maximum(_bdot(h0, w1_ref[...]) + b1_ref[...], 0.0)
    h2 = _bdot(h1, w2_ref[...]) + b2_ref[...]
    out_ref[...] = x + _ln_affine(h2, g_ref[...], beta_ref[...])


def _node_update(node_x, ps, wa, wb, w1, w2, b0, b1, b2, g, beta):
    bn = 1000
    grid = (N // bn,)
    row = lambda i: (i, 0)
    fix = lambda i: (0, 0)
    return pl.pallas_call(
        _node_update_body,
        grid=grid,
        in_specs=[pl.BlockSpec((bn, D), row)] * 3 + [
            pl.BlockSpec((D, D), fix),
            pl.BlockSpec((D, D), fix),
            pl.BlockSpec((D, D), fix),
            pl.BlockSpec((D, D), fix),
            pl.BlockSpec((1, D), fix),
            pl.BlockSpec((1, D), fix),
            pl.BlockSpec((1, D), fix),
            pl.BlockSpec((1, D), fix),
            pl.BlockSpec((1, D), fix),
        ],
        out_specs=pl.BlockSpec((bn, D), row),
        out_shape=jax.ShapeDtypeStruct((N, D), jnp.float32),
    )(node_x, *ps, wa, wb, w1, w2, b0, b1, b2, g, beta)


@functools.cache
def _sc_gather_kernel(half):
    mesh = plsc.VectorSubcoreMesh(core_axis_name="c", subcore_axis_name="s",
                                  num_cores=NC, num_subcores=NS)

    @functools.partial(
        pl.kernel,
        out_type=jax.ShapeDtypeStruct((E2, D), jnp.float32),
        mesh=mesh,
        scratch_types=[
            pltpu.VMEM((EPW,), jnp.int32),
            pltpu.VMEM((EPW,), jnp.int32),
            pltpu.VMEM((CHUNK, D), jnp.float32),
            pltpu.VMEM((CHUNK, D), jnp.float32),
            pltpu.VMEM((CHUNK, D), jnp.float32),
            pltpu.SemaphoreType.DMA,
            pltpu.SemaphoreType.DMA,
            pltpu.SemaphoreType.DMA,
            pltpu.SemaphoreType.DMA,
            pltpu.SemaphoreType.DMA,
            pltpu.SemaphoreType.DMA,
            pltpu.SemaphoreType.DMA,
            pltpu.SemaphoreType.DMA,
            pltpu.SemaphoreType.DMA,
        ],
    )
    def _sc_gather(pa_hbm, pb_hbm, dst_hbm, src_hbm, gsum_hbm,
                   idx_a, idx_b, rows0, rows1, rows2,
                   as0, as1, as2, bs0, bs1, bs2, ws0, ws1, ws2):
        wid = lax.axis_index("s") * NC + lax.axis_index("c")
        base0 = half * E2 + wid * EPW
        obase0 = wid * EPW
        rows_v = (rows0, rows1, rows2)
        asem = (as0, as1, as2)
        bsem = (bs0, bs1, bs2)
        wsem = (ws0, ws1, ws2)
        pltpu.sync_copy(dst_hbm.at[pl.ds(base0, EPW)], idx_a)
        pltpu.sync_copy(src_hbm.at[pl.ds(base0, EPW)], idx_b)
        pend_a = {}
        pend_b = {}
        pend_w = {}

        def start_a(j):
            pend_a[j] = pltpu.async_copy(
                pa_hbm.at[idx_a.at[pl.ds(j * CHUNK, CHUNK)]],
                rows_v[j % 3], asem[j % 3])

        def start_b(j):
            pend_a.pop(j).wait()
            pend_b[j] = pltpu.async_copy(
                pb_hbm.at[idx_b.at[pl.ds(j * CHUNK, CHUNK)]],
                rows_v[j % 3], bsem[j % 3], add=True)

        def start_w(j):
            pend_b.pop(j).wait()
            pend_w[j] = pltpu.async_copy(
                rows_v[j % 3],
                gsum_hbm.at[pl.ds(obase0 + j * CHUNK, CHUNK)], wsem[j % 3])

        for j in range(NCHUNK):
            if j >= 3:
                pend_w.pop(j - 3).wait()
            start_a(j)
            if j >= 1:
                start_b(j - 1)
            if j >= 2:
                start_w(j - 2)
        start_b(NCHUNK - 1)
        start_w(NCHUNK - 2)
        start_w(NCHUNK - 1)
        for j in (NCHUNK - 3, NCHUNK - 2, NCHUNK - 1):
            if j >= 0 and j in pend_w:
                pend_w.pop(j).wait()

    return _sc_gather


NPT = -(-((N + NS - 1) // NS) // 8) * 8
NPAD = NPT * NS
EPW_S = E // NW
CHUNK_S = 80
NCHUNK_S = EPW_S // CHUNK_S


@functools.cache
def _sc_scatter_add_kernel():
    mesh = plsc.VectorSubcoreMesh(core_axis_name="c", subcore_axis_name="s",
                                  num_cores=NC, num_subcores=NS)

    @functools.partial(
        pl.kernel,
        out_type=jax.ShapeDtypeStruct((NC, NPAD, D), jnp.float32),
        mesh=mesh,
        scratch_types=[
            pltpu.VMEM((NCHUNK_S, CHUNK_S), jnp.int32),
            pltpu.VMEM((CHUNK_S, D), jnp.float32),
            pltpu.VMEM((CHUNK_S, D), jnp.float32),
            pltpu.VMEM((CHUNK_S, D), jnp.float32),
            pltpu.VMEM_SHARED((NPAD, D), jnp.float32),
            pltpu.SemaphoreType.DMA,
            pltpu.SemaphoreType.DMA,
            pltpu.SemaphoreType.DMA,
            pltpu.SemaphoreType.DMA,
            pltpu.SemaphoreType.DMA,
            pltpu.SemaphoreType.DMA,
        ],
    )
    def _sc_scatter_add(edge0_hbm, edge1_hbm, dst3_hbm, zeros_hbm, out_hbm,
                        idx_all, rows0, rows1, rows2, agg_sh,
                        ls0, ls1, ls2, as0, as1, as2):
        cid = lax.axis_index("c")
        sid = lax.axis_index("s")
        wid = cid * NS + sid
        base0 = sid * EPW_S
        pltpu.sync_copy(dst3_hbm.at[wid], idx_all)
        pltpu.sync_copy(zeros_hbm.at[pl.ds(sid * NPT, NPT)],
                        agg_sh.at[pl.ds(sid * NPT, NPT)])
        plsc.subcore_barrier()
        rows_v = (rows0, rows1, rows2)
        lsem = (ls0, ls1, ls2)
        addsem = (as0, as1, as2)

        def run_half(edge_hbm):
            pend_l = {}
            pend_add = {}

            def start_l(c):
                pend_l[c] = pltpu.async_copy(
                    edge_hbm.at[pl.ds(base0 + c * CHUNK_S, CHUNK_S)],
                    rows_v[c % 3], lsem[c % 3])

            start_l(0)
            for c in range(NCHUNK_S):
                if c + 1 < NCHUNK_S:
                    if c >= 2:
                        pend_add.pop(c - 2).wait()
                    start_l(c + 1)
                pend_l.pop(c).wait()
                pend_add[c] = pltpu.async_copy(
                    rows_v[c % 3], agg_sh.at[idx_all.at[c]], addsem[c % 3],
                    add=True)
            for c in sorted(pend_add):
                pend_add.pop(c).wait()

        @pl.when(cid == 0)
        def _():
            run_half(edge0_hbm)

        @pl.when(cid == 1)
        def _():
            run_half(edge1_hbm)

        plsc.subcore_barrier()
        pltpu.sync_copy(agg_sh.at[pl.ds(sid * NPT, NPT)],
                        out_hbm.at[cid, pl.ds(sid * NPT, NPT)])

    return _sc_scatter_add


def kernel(node_x, edge_index, edge_attr,
           mW0, mb0, mW1, mb1, mW2, mb2, mg, mB,
           uW0, ub0, uW1, ub1, uW2, ub2, ug, uB):
    src = edge_index[0]
    dst = edge_index[1]

    pa, pb = _node_proj(node_x, mW0[:D], mW0[D:2 * D])
    zeros = jnp.zeros((NPAD, D), jnp.float32)
    dst3s = dst.reshape(NW, NCHUNK_S, CHUNK_S)
    eb = (mb0.reshape(1, D), mb1.reshape(1, D), mb2.reshape(1, D),
          mg.reshape(1, D), mB.reshape(1, D))

    halves = []
    for h in range(NH):
        gsum = _sc_gather_kernel(h)(pa, pb, dst, src)
        halves.append(
            _edge_mlp(gsum, edge_attr, h, mW0[2 * D:], mW1, mW2, *eb))
    partials = _sc_scatter_add_kernel()(halves[0], halves[1], dst3s, zeros)

    ps = [partials[c, :N] for c in range(NC)]
    out = _node_update(
        node_x, ps, uW0[:D], uW0[D:], uW1, uW2,
        ub0.reshape(1, D), ub1.reshape(1, D), ub2.reshape(1, D),
        ug.reshape(1, D), uB.reshape(1, D))
    return out


# speedup vs baseline: 1.3712x; 1.0365x over previous
import functools

import jax
import jax.numpy as jnp
from jax import lax
from jax.experimental import pallas as pl
from jax.experimental.pallas import tpu as pltpu
from jax.experimental.pallas import tpu_sc as plsc

N = 10000
E = 320000
D = 128
NH = 2
E2 = E // NH

NC = 2
NS = 16
NW = NC * NS
EPW = E2 // NW
CHUNK = 200
NCHUNK = EPW // CHUNK


def _bdot(a, b):
    return jnp.dot(a.astype(jnp.bfloat16), b.astype(jnp.bfloat16),
                   preferred_element_type=jnp.float32)


def _node_proj_body(x_ref, wa_ref, wb_ref, pa_ref, pb_ref):
    x = x_ref[...]
    pa_ref[...] = _bdot(x, wa_ref[...])
    pb_ref[...] = _bdot(x, wb_ref[...])


def _node_proj(node_x, wa, wb):
    bn = 1000
    grid = (N // bn,)
    return pl.pallas_call(
        _node_proj_body,
        grid=grid,
        in_specs=[
            pl.BlockSpec((bn, D), lambda i: (i, 0)),
            pl.BlockSpec((D, D), lambda i: (0, 0)),
            pl.BlockSpec((D, D), lambda i: (0, 0)),
        ],
        out_specs=[
            pl.BlockSpec((bn, D), lambda i: (i, 0)),
            pl.BlockSpec((bn, D), lambda i: (i, 0)),
        ],
        out_shape=[
            jax.ShapeDtypeStruct((N, D), jnp.float32),
            jax.ShapeDtypeStruct((N, D), jnp.float32),
        ],
    )(node_x, wa, wb)


def _ln_affine(h, g, b):
    mu = jnp.mean(h, axis=-1, keepdims=True)
    hc = h - mu
    var = jnp.mean(hc * hc, axis=-1, keepdims=True)
    return hc * lax.rsqrt(var + 1e-5) * g + b


def _edge_mlp_body(gsum_ref, e_ref, wc_ref, w1_ref, w2_ref,
                   b0_ref, b1_ref, b2_ref, g_ref, beta_ref, out_ref):
    e = e_ref[...]
    h0 = gsum_ref[...] + _bdot(e, wc_ref[...]) + b0_ref[...]
    h0 = jnp.maximum(h0, 0.0)
    h1 = jnp.maximum(_bdot(h0, w1_ref[...]) + b1_ref[...], 0.0)
    h2 = _bdot(h1, w2_ref[...]) + b2_ref[...]
    out_ref[...] = e + _ln_affine(h2, g_ref[...], beta_ref[...])


BE = 6400


def _edge_mlp(gsum, edge_attr, half, wc, w1, w2, b0, b1, b2, g, beta):
    grid = (E2 // BE,)
    off = half * (E2 // BE)
    row = lambda i: (i, 0)
    erow = lambda i: (i + off, 0)
    fix = lambda i: (0, 0)
    return pl.pallas_call(
        _edge_mlp_body,
        grid=grid,
        in_specs=[
            pl.BlockSpec((BE, D), row),
            pl.BlockSpec((BE, D), erow),
            pl.BlockSpec((D, D), fix),
            pl.BlockSpec((D, D), fix),
            pl.BlockSpec((D, D), fix),
            pl.BlockSpec((1, D), fix),
            pl.BlockSpec((1, D), fix),
            pl.BlockSpec((1, D), fix),
            pl.BlockSpec((1, D), fix),
            pl.BlockSpec((1, D), fix),
        ],
        out_specs=pl.BlockSpec((BE, D), row),
        out_shape=jax.ShapeDtypeStruct((E2, D), jnp.float32),
    )(gsum, edge_attr, wc, w1, w2, b0, b1, b2, g, beta)


def _node_update_body(x_ref, p0_ref, p1_ref,
                      wa_ref, wb_ref, w1_ref, w2_ref,
                      b0_ref, b1_ref, b2_ref, g_ref, beta_ref, out_ref):
    x = x_ref[...]
    agg = p0_ref[...] + p1_ref[...]
    h0 = _bdot(x, wa_ref[...]) + _bdot(agg, wb_ref[...]) + b0_ref[...]
    h0 = jnp.maximum(h0, 0.0)
    h1 = jnp.maximum(_bdot(h0, w1_ref[...]) + b1_ref[...], 0.0)
    h2 = _bdot(h1, w2_ref[...]) + b2_ref[...]
    out_ref[...] = x + _ln_affine(h2, g_ref[...], beta_ref[...])


def _node_update(node_x, ps, wa, wb, w1, w2, b0, b1, b2, g, beta):
    bn = 1000
    grid = (N // bn,)
    row = lambda i: (i, 0)
    fix = lambda i: (0, 0)
    return pl.pallas_call(
        _node_update_body,
        grid=grid,
        in_specs=[pl.BlockSpec((bn, D), row)] * 3 + [
            pl.BlockSpec((D, D), fix),
            pl.BlockSpec((D, D), fix),
            pl.BlockSpec((D, D), fix),
            pl.BlockSpec((D, D), fix),
            pl.BlockSpec((1, D), fix),
            pl.BlockSpec((1, D), fix),
            pl.BlockSpec((1, D), fix),
            pl.BlockSpec((1, D), fix),
            pl.BlockSpec((1, D), fix),
        ],
        out_specs=pl.BlockSpec((bn, D), row),
        out_shape=jax.ShapeDtypeStruct((N, D), jnp.float32),
    )(node_x, *ps, wa, wb, w1, w2, b0, b1, b2, g, beta)


@functools.cache
def _sc_gather_kernel(half):
    mesh = plsc.VectorSubcoreMesh(core_axis_name="c", subcore_axis_name="s",
                                  num_cores=NC, num_subcores=NS)

    @functools.partial(
        pl.kernel,
        out_type=jax.ShapeDtypeStruct((E2, D), jnp.float32),
        mesh=mesh,
        scratch_types=[
            pltpu.VMEM((EPW,), jnp.int32),
            pltpu.VMEM((EPW,), jnp.int32),
            pltpu.VMEM((CHUNK, D), jnp.float32),
            pltpu.VMEM((CHUNK, D), jnp.float32),
            pltpu.VMEM((CHUNK, D), jnp.float32),
            pltpu.SemaphoreType.DMA,
            pltpu.SemaphoreType.DMA,
            pltpu.SemaphoreType.DMA,
            pltpu.SemaphoreType.DMA,
            pltpu.SemaphoreType.DMA,
            pltpu.SemaphoreType.DMA,
            pltpu.SemaphoreType.DMA,
            pltpu.SemaphoreType.DMA,
            pltpu.SemaphoreType.DMA,
        ],
    )
    def _sc_gather(pa_hbm, pb_hbm, dst_hbm, src_hbm, gsum_hbm,
                   idx_a, idx_b, rows0, rows1, rows2,
                   as0, as1, as2, bs0, bs1, bs2, ws0, ws1, ws2):
        wid = lax.axis_index("s") * NC + lax.axis_index("c")
        base0 = half * E2 + wid * EPW
        obase0 = wid * EPW
        rows_v = (rows0, rows1, rows2)
        asem = (as0, as1, as2)
        bsem = (bs0, bs1, bs2)
        wsem = (ws0, ws1, ws2)
        pltpu.sync_copy(dst_hbm.at[pl.ds(base0, EPW)], idx_a)
        pltpu.sync_copy(src_hbm.at[pl.ds(base0, EPW)], idx_b)
        pend_a = {}
        pend_b = {}
        pend_w = {}

        def start_a(j):
            pend_a[j] = pltpu.async_copy(
                pa_hbm.at[idx_a.at[pl.ds(j * CHUNK, CHUNK)]],
                rows_v[j % 3], asem[j % 3])

        def start_b(j):
            pend_a.pop(j).wait()
            pend_b[j] = pltpu.async_copy(
                pb_hbm.at[idx_b.at[pl.ds(j * CHUNK, CHUNK)]],
                rows_v[j % 3], bsem[j % 3], add=True)

        def start_w(j):
            pend_b.pop(j).wait()
            pend_w[j] = pltpu.async_copy(
                rows_v[j % 3],
                gsum_hbm.at[pl.ds(obase0 + j * CHUNK, CHUNK)], wsem[j % 3])

        for j in range(NCHUNK):
            if j >= 3:
                pend_w.pop(j - 3).wait()
            start_a(j)
            if j >= 1:
                start_b(j - 1)
            if j >= 2:
                start_w(j - 2)
        start_b(NCHUNK - 1)
        start_w(NCHUNK - 2)
        start_w(NCHUNK - 1)
        for j in (NCHUNK - 3, NCHUNK - 2, NCHUNK - 1):
            if j >= 0 and j in pend_w:
                pend_w.pop(j).wait()

    return _sc_gather


NPT = -(-((N + NS - 1) // NS) // 8) * 8
NPAD = NPT * NS
EPW_S = E // NW
CHUNK_S = 80
NCHUNK_S = EPW_S // CHUNK_S


@functools.cache
def _sc_scatter_add_kernel():
    mesh = plsc.VectorSubcoreMesh(core_axis_name="c", subcore_axis_name="s",
                                  num_cores=NC, num_subcores=NS)

    @functools.partial(
        pl.kernel,
        out_type=jax.ShapeDtypeStruct((NC, NPAD, D), jnp.float32),
        mesh=mesh,
        scratch_types=[
            pltpu.VMEM((NCHUNK_S, CHUNK_S), jnp.int32),
            pltpu.VMEM((CHUNK_S, D), jnp.float32),
            pltpu.VMEM((CHUNK_S, D), jnp.float32),
            pltpu.VMEM((CHUNK_S, D), jnp.float32),
            pltpu.VMEM_SHARED((NPAD, D), jnp.float32),
            pltpu.SemaphoreType.DMA,
            pltpu.SemaphoreType.DMA,
            pltpu.SemaphoreType.DMA,
            pltpu.SemaphoreType.DMA,
            pltpu.SemaphoreType.DMA,
            pltpu.SemaphoreType.DMA,
        ],
    )
    def _sc_scatter_add(edge0_hbm, edge1_hbm, dst3_hbm, zeros_hbm, out_hbm,
                        idx_all, rows0, rows1, rows2, agg_sh,
                        ls0, ls1, ls2, as0, as1, as2):
        cid = lax.axis_index("c")
        sid = lax.axis_index("s")
        wid = cid * NS + sid
        base0 = sid * EPW_S
        pltpu.sync_copy(dst3_hbm.at[wid], idx_all)
        pltpu.sync_copy(zeros_hbm.at[pl.ds(sid * NPT, NPT)],
                        agg_sh.at[pl.ds(sid * NPT, NPT)])
        plsc.subcore_barrier()
        rows_v = (rows0, rows1, rows2)
        lsem = (ls0, ls1, ls2)
        addsem = (as0, as1, as2)

        def run_half(edge_hbm):
            pend_l = {}
            pend_add = {}

            def start_l(c):
                pend_l[c] = pltpu.async_copy(
                    edge_hbm.at[pl.ds(base0 + c * CHUNK_S, CHUNK_S)],
                    rows_v[c % 3], lsem[c % 3])

            start_l(0)
            for c in range(NCHUNK_S):
                if c + 1 < NCHUNK_S:
                    if c >= 2:
                        pend_add.pop(c - 2).wait()
                    start_l(c + 1)
                pend_l.pop(c).wait()
                pend_add[c] = pltpu.async_copy(
                    rows_v[c % 3], agg_sh.at[idx_all.at[c]], addsem[c % 3],
                    add=True)
            for c in sorted(pend_add):
                pend_add.pop(c).wait()

        @pl.when(cid == 0)
        def _():
            run_half(edge0_hbm)

        @pl.when(cid == 1)
        def _():
            run_half(edge1_hbm)

        plsc.subcore_barrier()
        pltpu.sync_copy(agg_sh.at[pl.ds(sid * NPT, NPT)],
                        out_hbm.at[cid, pl.ds(sid * NPT, NPT)])

    return _sc_scatter_add


def kernel(node_x, edge_index, edge_attr,
           mW0, mb0, mW1, mb1, mW2, mb2, mg, mB,
           uW0, ub0, uW1, ub1, uW2, ub2, ug, uB):
    src = edge_index[0]
    dst = edge_index[1]

    pa, pb = _node_proj(node_x, mW0[:D], mW0[D:2 * D])
    zeros = jnp.zeros((NPAD, D), jnp.float32)
    dst3s = dst.reshape(NW, NCHUNK_S, CHUNK_S)
    eb = (mb0.reshape(1, D), mb1.reshape(1, D), mb2.reshape(1, D),
          mg.reshape(1, D), mB.reshape(1, D))

    halves = []
    for h in range(NH):
        gsum = _sc_gather_kernel(h)(pa, pb, dst, src)
        halves.append(
            _edge_mlp(gsum, edge_attr, h, mW0[2 * D:], mW1, mW2, *eb))
    partials = _sc_scatter_add_kernel()(halves[0], halves[1], dst3s, zeros)

    ps = [partials[c, :N] for c in range(NC)]
    out = _node_update(
        node_x, ps, uW0[:D], uW0[D:], uW1, uW2,
        ub0.reshape(1, D), ub1.reshape(1, D), ub2.reshape(1, D),
        ug.reshape(1, D), uB.reshape(1, D))
    return out


# speedup vs baseline: 1.3795x; 1.0060x over previous
import functools

import jax
import jax.numpy as jnp
from jax import lax
from jax.experimental import pallas as pl
from jax.experimental.pallas import tpu as pltpu
from jax.experimental.pallas import tpu_sc as plsc

N = 10000
E = 320000
D = 128
NH = 2
E2 = E // NH

NC = 2
NS = 16
NW = NC * NS
EPW = E2 // NW
CHUNK = 200
NCHUNK = EPW // CHUNK


def _bdot(a, b):
    return jnp.dot(a.astype(jnp.bfloat16), b.astype(jnp.bfloat16),
                   preferred_element_type=jnp.float32)


def _node_proj_body(x_ref, wa_ref, wb_ref, pa_ref, pb_ref):
    x = x_ref[...]
    pa_ref[...] = _bdot(x, wa_ref[...])
    pb_ref[...] = _bdot(x, wb_ref[...])


def _node_proj(node_x, wa, wb):
    bn = 1000
    grid = (N // bn,)
    return pl.pallas_call(
        _node_proj_body,
        grid=grid,
        in_specs=[
            pl.BlockSpec((bn, D), lambda i: (i, 0)),
            pl.BlockSpec((D, D), lambda i: (0, 0)),
            pl.BlockSpec((D, D), lambda i: (0, 0)),
        ],
        out_specs=[
            pl.BlockSpec((bn, D), lambda i: (i, 0)),
            pl.BlockSpec((bn, D), lambda i: (i, 0)),
        ],
        out_shape=[
            jax.ShapeDtypeStruct((N, D), jnp.float32),
            jax.ShapeDtypeStruct((N, D), jnp.float32),
        ],
    )(node_x, wa, wb)


def _ln_affine(h, g, b):
    mu = jnp.mean(h, axis=-1, keepdims=True)
    hc = h - mu
    var = jnp.mean(hc * hc, axis=-1, keepdims=True)
    return hc * lax.rsqrt(var + 1e-5) * g + b


def _edge_mlp_body(gsum_ref, e_ref, wc_ref, w1_ref, w2_ref,
                   b0_ref, b1_ref, b2_ref, g_ref, beta_ref, out_ref):
    e = e_ref[...]
    h0 = gsum_ref[...] + _bdot(e, wc_ref[...]) + b0_ref[...]
    h0 = jnp.maximum(h0, 0.0)
    h1 = jnp.maximum(_bdot(h0, w1_ref[...]) + b1_ref[...], 0.0)
    h2 = _bdot(h1, w2_ref[...]) + b2_ref[...]
    out_ref[...] = e + _ln_affine(h2, g_ref[...], beta_ref[...])


BE = 8000


def _edge_mlp(gsum, edge_attr, half, wc, w1, w2, b0, b1, b2, g, beta):
    grid = (E2 // BE,)
    off = half * (E2 // BE)
    row = lambda i: (i, 0)
    erow = lambda i: (i + off, 0)
    fix = lambda i: (0, 0)
    return pl.pallas_call(
        _edge_mlp_body,
        grid=grid,
        in_specs=[
            pl.BlockSpec((BE, D), row),
            pl.BlockSpec((BE, D), erow),
            pl.BlockSpec((D, D), fix),
            pl.BlockSpec((D, D), fix),
            pl.BlockSpec((D, D), fix),
            pl.BlockSpec((1, D), fix),
            pl.BlockSpec((1, D), fix),
            pl.BlockSpec((1, D), fix),
            pl.BlockSpec((1, D), fix),
            pl.BlockSpec((1, D), fix),
        ],
        out_specs=pl.BlockSpec((BE, D), row),
        out_shape=jax.ShapeDtypeStruct((E2, D), jnp.float32),
    )(gsum, edge_attr, wc, w1, w2, b0, b1, b2, g, beta)


def _node_update_body(x_ref, p0_ref, p1_ref,
                      wa_ref, wb_ref, w1_ref, w2_ref,
                      b0_ref, b1_ref, b2_ref, g_ref, beta_ref, out_ref):
    x = x_ref[...]
    agg = p0_ref[...] + p1_ref[...]
    h0 = _bdot(x, wa_ref[...]) + _bdot(agg, wb_ref[...]) + b0_ref[...]
    h0 = jnp.maximum(h0, 0.0)
    h1 = jnp.maximum(_bdot(h0, w1_ref[...]) + b1_ref[...], 0.0)
    h2 = _bdot(h1, w2_ref[...]) + b2_ref[...]
    out_ref[...] = x + _ln_affine(h2, g_ref[...], beta_ref[...])


def _node_update(node_x, ps, wa, wb, w1, w2, b0, b1, b2, g, beta):
    bn = 1000
    grid = (N // bn,)
    row = lambda i: (i, 0)
    fix = lambda i: (0, 0)
    return pl.pallas_call(
        _node_update_body,
        grid=grid,
        in_specs=[pl.BlockSpec((bn, D), row)] * 3 + [
            pl.BlockSpec((D, D), fix),
            pl.BlockSpec((D, D), fix),
            pl.BlockSpec((D, D), fix),
            pl.BlockSpec((D, D), fix),
            pl.BlockSpec((1, D), fix),
            pl.BlockSpec((1, D), fix),
            pl.BlockSpec((1, D), fix),
            pl.BlockSpec((1, D), fix),
            pl.BlockSpec((1, D), fix),
        ],
        out_specs=pl.BlockSpec((bn, D), row),
        out_shape=jax.ShapeDtypeStruct((N, D), jnp.float32),
    )(node_x, *ps, wa, wb, w1, w2, b0, b1, b2, g, beta)


@functools.cache
def _sc_gather_kernel(half):
    mesh = plsc.VectorSubcoreMesh(core_axis_name="c", subcore_axis_name="s",
                                  num_cores=NC, num_subcores=NS)

    @functools.partial(
        pl.kernel,
        out_type=jax.ShapeDtypeStruct((E2, D), jnp.float32),
        mesh=mesh,
        scratch_types=[
            pltpu.VMEM((EPW,), jnp.int32),
            pltpu.VMEM((EPW,), jnp.int32),
            pltpu.VMEM((CHUNK, D), jnp.float32),
            pltpu.VMEM((CHUNK, D), jnp.float32),
            pltpu.VMEM((CHUNK, D), jnp.float32),
            pltpu.SemaphoreType.DMA,
            pltpu.SemaphoreType.DMA,
            pltpu.SemaphoreType.DMA,
            pltpu.SemaphoreType.DMA,
            pltpu.SemaphoreType.DMA,
            pltpu.SemaphoreType.DMA,
            pltpu.SemaphoreType.DMA,
            pltpu.SemaphoreType.DMA,
            pltpu.SemaphoreType.DMA,
        ],
    )
    def _sc_gather(pa_hbm, pb_hbm, dst_hbm, src_hbm, gsum_hbm,
                   idx_a, idx_b, rows0, rows1, rows2,
                   as0, as1, as2, bs0, bs1, bs2, ws0, ws1, ws2):
        wid = lax.axis_index("s") * NC + lax.axis_index("c")
        base0 = half * E2 + wid * EPW
        obase0 = wid * EPW
        rows_v = (rows0, rows1, rows2)
        asem = (as0, as1, as2)
        bsem = (bs0, bs1, bs2)
        wsem = (ws0, ws1, ws2)
        pltpu.sync_copy(dst_hbm.at[pl.ds(base0, EPW)], idx_a)
        pltpu.sync_copy(src_hbm.at[pl.ds(base0, EPW)], idx_b)
        pend_a = {}
        pend_b = {}
        pend_w = {}

        def start_a(j):
            pend_a[j] = pltpu.async_copy(
                pa_hbm.at[idx_a.at[pl.ds(j * CHUNK, CHUNK)]],
                rows_v[j % 3], asem[j % 3])

        def start_b(j):
            pend_a.pop(j).wait()
            pend_b[j] = pltpu.async_copy(
                pb_hbm.at[idx_b.at[pl.ds(j * CHUNK, CHUNK)]],
                rows_v[j % 3], bsem[j % 3], add=True)

        def start_w(j):
            pend_b.pop(j).wait()
            pend_w[j] = pltpu.async_copy(
                rows_v[j % 3],
                gsum_hbm.at[pl.ds(obase0 + j * CHUNK, CHUNK)], wsem[j % 3])

        for j in range(NCHUNK):
            if j >= 3:
                pend_w.pop(j - 3).wait()
            start_a(j)
            if j >= 1:
                start_b(j - 1)
            if j >= 2:
                start_w(j - 2)
        start_b(NCHUNK - 1)
        start_w(NCHUNK - 2)
        start_w(NCHUNK - 1)
        for j in (NCHUNK - 3, NCHUNK - 2, NCHUNK - 1):
            if j >= 0 and j in pend_w:
                pend_w.pop(j).wait()

    return _sc_gather


NPT = -(-((N + NS - 1) // NS) // 8) * 8
NPAD = NPT * NS
EPW_S = E // NW
CHUNK_S = 80
NCHUNK_S = EPW_S // CHUNK_S


@functools.cache
def _sc_scatter_add_kernel():
    mesh = plsc.VectorSubcoreMesh(core_axis_name="c", subcore_axis_name="s",
                                  num_cores=NC, num_subcores=NS)

    @functools.partial(
        pl.kernel,
        out_type=jax.ShapeDtypeStruct((NC, NPAD, D), jnp.float32),
        mesh=mesh,
        scratch_types=[
            pltpu.VMEM((NCHUNK_S, CHUNK_S), jnp.int32),
            pltpu.VMEM((CHUNK_S, D), jnp.float32),
            pltpu.VMEM((CHUNK_S, D), jnp.float32),
            pltpu.VMEM((CHUNK_S, D), jnp.float32),
            pltpu.VMEM_SHARED((NPAD, D), jnp.float32),
            pltpu.SemaphoreType.DMA,
            pltpu.SemaphoreType.DMA,
            pltpu.SemaphoreType.DMA,
            pltpu.SemaphoreType.DMA,
            pltpu.SemaphoreType.DMA,
            pltpu.SemaphoreType.DMA,
        ],
    )
    def _sc_scatter_add(edge0_hbm, edge1_hbm, dst3_hbm, zeros_hbm, out_hbm,
                        idx_all, rows0, rows1, rows2, agg_sh,
                        ls0, ls1, ls2, as0, as1, as2):
        cid = lax.axis_index("c")
        sid = lax.axis_index("s")
        wid = cid * NS + sid
        base0 = sid * EPW_S
        pltpu.sync_copy(dst3_hbm.at[wid], idx_all)
        pltpu.sync_copy(zeros_hbm.at[pl.ds(sid * NPT, NPT)],
                        agg_sh.at[pl.ds(sid * NPT, NPT)])
        plsc.subcore_barrier()
        rows_v = (rows0, rows1, rows2)
        lsem = (ls0, ls1, ls2)
        addsem = (as0, as1, as2)

        def run_half(edge_hbm):
            pend_l = {}
            pend_add = {}

            def start_l(c):
                pend_l[c] = pltpu.async_copy(
                    edge_hbm.at[pl.ds(base0 + c * CHUNK_S, CHUNK_S)],
                    rows_v[c % 3], lsem[c % 3])

            start_l(0)
            for c in range(NCHUNK_S):
                if c + 1 < NCHUNK_S:
                    if c >= 2:
                        pend_add.pop(c - 2).wait()
                    start_l(c + 1)
                pend_l.pop(c).wait()
                pend_add[c] = pltpu.async_copy(
                    rows_v[c % 3], agg_sh.at[idx_all.at[c]], addsem[c % 3],
                    add=True)
            for c in sorted(pend_add):
                pend_add.pop(c).wait()

        @pl.when(cid == 0)
        def _():
            run_half(edge0_hbm)

        @pl.when(cid == 1)
        def _():
            run_half(edge1_hbm)

        plsc.subcore_barrier()
        pltpu.sync_copy(agg_sh.at[pl.ds(sid * NPT, NPT)],
                        out_hbm.at[cid, pl.ds(sid * NPT, NPT)])

    return _sc_scatter_add


def kernel(node_x, edge_index, edge_attr,
           mW0, mb0, mW1, mb1, mW2, mb2, mg, mB,
           uW0, ub0, uW1, ub1, uW2, ub2, ug, uB):
    src = edge_index[0]
    dst = edge_index[1]

    pa, pb = _node_proj(node_x, mW0[:D], mW0[D:2 * D])
    zeros = jnp.zeros((NPAD, D), jnp.float32)
    dst3s = dst.reshape(NW, NCHUNK_S, CHUNK_S)
    eb = (mb0.reshape(1, D), mb1.reshape(1, D), mb2.reshape(1, D),
          mg.reshape(1, D), mB.reshape(1, D))

    halves = []
    for h in range(NH):
        gsum = _sc_gather_kernel(h)(pa, pb, dst, src)
        halves.append(
            _edge_mlp(gsum, edge_attr, h, mW0[2 * D:], mW1, mW2, *eb))
    partials = _sc_scatter_add_kernel()(halves[0], halves[1], dst3s, zeros)

    ps = [partials[c, :N] for c in range(NC)]
    out = _node_update(
        node_x, ps, uW0[:D], uW0[D:], uW1, uW2,
        ub0.reshape(1, D), ub1.reshape(1, D), ub2.reshape(1, D),
        ug.reshape(1, D), uB.reshape(1, D))
    return out
